# Initial kernel scaffold; baseline (speedup 1.0000x reference)
import jax
import jax.numpy as jnp
from jax.experimental import pallas as pl


def kernel(observations, g_W1, g_b1, g_W2, g_b2, g_W3, g_b3, e_W1, e_b1, e_W2, e_b2, e_W3, e_b3):
    raise NotImplementedError("write your pallas kernel here")


# speedup vs baseline: 1.2807x; 1.2807x over previous
import functools

import jax
import jax.numpy as jnp
from jax.experimental import pallas as pl
from jax.experimental.pallas import tpu as pltpu

N = 2048
D = 768
E = 64
A = 32


def _elu(x):
    return jnp.where(x > 0, x, jnp.exp(jnp.minimum(x, 0.0)) - 1.0)


def _gating_body(obs_ref, w1_ref, b1_ref, w2_ref, b2_ref, w3_ref, b3_ref,
                 wfull_ref):
    x = obs_ref[...]
    h = _elu(jnp.dot(x, w1_ref[...], preferred_element_type=jnp.float32)
             + b1_ref[...])
    h = _elu(jnp.dot(h, w2_ref[...], preferred_element_type=jnp.float32)
             + b2_ref[...])
    logits = (jnp.dot(h, w3_ref[...], preferred_element_type=jnp.float32)
              + b3_ref[...])
    iota = jax.lax.broadcasted_iota(jnp.int32, (N, E), 1)
    m1 = jnp.max(logits, axis=-1, keepdims=True)
    idx1 = jnp.min(jnp.where(logits == m1, iota, E + 1), axis=-1,
                   keepdims=True)
    masked = jnp.where(iota == idx1, -1e30, logits)
    m2 = jnp.max(masked, axis=-1, keepdims=True)
    idx2 = jnp.min(jnp.where(masked == m2, iota, E + 1), axis=-1,
                   keepdims=True)
    w0 = 1.0 / (1.0 + jnp.exp(m2 - m1))
    w1v = 1.0 - w0
    wfull_ref[...] = (jnp.where(iota == idx1, w0, 0.0)
                      + jnp.where(iota == idx2, w1v, 0.0))


def _dense_expert_body(obs_ref, wfull_ref, w1_ref, b1_ref, w2_ref, b2_ref,
                       w3_ref, b3_ref, acc_ref):
    e = pl.program_id(0)
    x = obs_ref[...]
    h = _elu(jnp.dot(x, w1_ref[0], preferred_element_type=jnp.float32)
             + b1_ref[0])
    h = _elu(jnp.dot(h, w2_ref[0], preferred_element_type=jnp.float32)
             + b2_ref[0])
    out = (jnp.dot(h, w3_ref[0], preferred_element_type=jnp.float32)
           + b3_ref[0])
    iota = jax.lax.broadcasted_iota(jnp.int32, (N, E), 1)
    wcol = jnp.sum(jnp.where(iota == e, wfull_ref[...], 0.0), axis=1,
                   keepdims=True)
    contrib = wcol * out

    @pl.when(e == 0)
    def _init():
        acc_ref[...] = contrib

    @pl.when(e > 0)
    def _acc():
        acc_ref[...] += contrib


def kernel(observations, g_W1, g_b1, g_W2, g_b2, g_W3, g_b3,
           e_W1, e_b1, e_W2, e_b2, e_W3, e_b3):
    wfull = pl.pallas_call(
        _gating_body,
        out_shape=jax.ShapeDtypeStruct((N, E), jnp.float32),
    )(observations, g_W1, g_b1.reshape(1, -1), g_W2, g_b2.reshape(1, -1),
      g_W3, g_b3.reshape(1, -1))

    grid_spec = pl.GridSpec(
        grid=(E,),
        in_specs=[
            pl.BlockSpec((N, D), lambda e: (0, 0)),
            pl.BlockSpec((N, E), lambda e: (0, 0)),
            pl.BlockSpec((1, D, 256), lambda e: (e, 0, 0)),
            pl.BlockSpec((1, 1, 256), lambda e: (e, 0, 0)),
            pl.BlockSpec((1, 256, 128), lambda e: (e, 0, 0)),
            pl.BlockSpec((1, 1, 128), lambda e: (e, 0, 0)),
            pl.BlockSpec((1, 128, A), lambda e: (e, 0, 0)),
            pl.BlockSpec((1, 1, A), lambda e: (e, 0, 0)),
        ],
        out_specs=pl.BlockSpec((N, A), lambda e: (0, 0)),
    )
    action_mean = pl.pallas_call(
        _dense_expert_body,
        grid_spec=grid_spec,
        out_shape=jax.ShapeDtypeStruct((N, A), jnp.float32),
        compiler_params=pltpu.CompilerParams(
            dimension_semantics=("arbitrary",),
        ),
    )(observations, wfull, e_W1, e_b1.reshape(E, 1, 256), e_W2,
      e_b2.reshape(E, 1, 128), e_W3, e_b3.reshape(E, 1, A))
    return action_mean
